# baseline (speedup 1.0000x reference)
import functools

import jax
import jax.numpy as jnp
from jax import lax
from jax.experimental import pallas as pl
from jax.experimental.pallas import tpu as pltpu
from jax.experimental.pallas import tpu_sc as plsc

N_FIELDS = 26
VOCAB = 100000
EMB = 16
BATCH = 4096
D = N_FIELDS * EMB
HID = 256
N_UNITS = 3

NC = 2
NS = 16
NW = NC * NS
R = BATCH * N_FIELDS
ROWS_PER_W = R // NW
CHUNK = 128
N_CHUNKS = ROWS_PER_W // CHUNK


def _sc_gather(flat_tables, idx3, out3_shape):
    mesh = plsc.VectorSubcoreMesh(core_axis_name="c", subcore_axis_name="s")

    @functools.partial(
        pl.kernel,
        out_type=jax.ShapeDtypeStruct(out3_shape, jnp.float32),
        mesh=mesh,
        scratch_types=[
            pltpu.VMEM((N_CHUNKS, CHUNK), jnp.int32),
            pltpu.VMEM((ROWS_PER_W, EMB), jnp.float32),
            pltpu.SemaphoreType.DMA,
        ],
        compiler_params=pltpu.CompilerParams(use_tc_tiling_on_sc=False),
    )
    def gather_kernel(table_hbm, idx_hbm, out_hbm, idx_v, rows_v, sem):
        wid = lax.axis_index("s") * NC + lax.axis_index("c")
        pltpu.sync_copy(idx_hbm.at[wid], idx_v)
        copies = []
        for j in range(N_CHUNKS):
            copies.append(
                pltpu.async_copy(
                    table_hbm.at[idx_v.at[j]],
                    rows_v.at[pl.ds(j * CHUNK, CHUNK)],
                    sem,
                )
            )
        for c in copies:
            c.wait()
        pltpu.sync_copy(rows_v, out_hbm.at[wid])

    return gather_kernel(flat_tables, idx3)


def _mlp_body(x_ref, w1_ref, b1_ref, w2_ref, b2_ref, wd_ref, bd_ref, o_ref):
    r = x_ref[...]
    for i in range(N_UNITS):
        h = jnp.dot(r, w1_ref[i], preferred_element_type=jnp.float32)
        h = jnp.maximum(h + b1_ref[i], 0.0)
        h = jnp.dot(h, w2_ref[i], preferred_element_type=jnp.float32)
        r = jnp.maximum(h + b2_ref[i] + r, 0.0)
    z = jnp.sum(r * wd_ref[...], axis=1, keepdims=True) + bd_ref[...]
    o_ref[...] = 1.0 / (1.0 + jnp.exp(-z))


def _mlp(emb, res_W1, res_b1, res_W2, res_b2, wd_row, bd11, block_b=512):
    grid = (BATCH // block_b,)
    return pl.pallas_call(
        _mlp_body,
        grid=grid,
        in_specs=[
            pl.BlockSpec((block_b, D), lambda i: (i, 0)),
            pl.BlockSpec((N_UNITS, D, HID), lambda i: (0, 0, 0)),
            pl.BlockSpec((N_UNITS, 1, HID), lambda i: (0, 0, 0)),
            pl.BlockSpec((N_UNITS, HID, D), lambda i: (0, 0, 0)),
            pl.BlockSpec((N_UNITS, 1, D), lambda i: (0, 0, 0)),
            pl.BlockSpec((1, D), lambda i: (0, 0)),
            pl.BlockSpec((1, 1), lambda i: (0, 0)),
        ],
        out_specs=pl.BlockSpec((block_b, 1), lambda i: (i, 0)),
        out_shape=jax.ShapeDtypeStruct((BATCH, 1), jnp.float32),
    )(emb, res_W1, res_b1, res_W2, res_b2, wd_row, bd11)


def kernel(inputs, tables, res_W1, res_b1, res_W2, res_b2, Wd, bd):
    flat_tables = tables.reshape(N_FIELDS * VOCAB, EMB)
    offs = (jnp.arange(N_FIELDS, dtype=jnp.int32) * VOCAB)[None, :]
    flat_idx = (inputs.astype(jnp.int32) + offs).reshape(NW, N_CHUNKS, CHUNK)
    rows = _sc_gather(flat_tables, flat_idx, (NW, ROWS_PER_W, EMB))
    emb = rows.reshape(BATCH, D)
    out = _mlp(
        emb,
        res_W1,
        res_b1.reshape(N_UNITS, 1, HID),
        res_W2,
        res_b2.reshape(N_UNITS, 1, D),
        Wd.reshape(1, D),
        bd.reshape(1, 1),
    )
    return out
